# baseline (speedup 1.0000x reference)
import jax
import jax.numpy as jnp
import numpy as np
from jax.experimental import pallas as pl
from jax.experimental.pallas import tpu as pltpu

_BR = 1024
_BC = 2048
_SUB = 128
_K = _BC // _SUB

_M = np.concatenate(
    [np.triu(np.ones((_SUB, _SUB), np.float32)),
     np.ones((_SUB, _SUB), np.float32)], axis=1)


def _body(x_ref, m_ref, o_ref, carry_ref):
    c = pl.program_id(0)

    @pl.when(c == 0)
    def _():
        carry_ref[...] = jnp.zeros_like(carry_ref)

    t = x_ref[...]
    m = m_ref[...]
    off = carry_ref[...]
    for i in range(_K):
        sub = t[:, i * _SUB:(i + 1) * _SUB]
        r = jax.lax.dot(
            sub, m,
            precision=jax.lax.Precision.DEFAULT,
            preferred_element_type=jnp.float32,
        )
        o_ref[:, i * _SUB:(i + 1) * _SUB] = r[:, :_SUB] + off
        off = off + r[:, _SUB:]

    @pl.when(c < pl.num_programs(0) - 1)
    def _():
        carry_ref[...] = off


@jax.jit
def kernel(x):
    R, C = x.shape
    m = jnp.asarray(_M)
    grid = (C // _BC,)
    return pl.pallas_call(
        _body,
        grid=grid,
        in_specs=[
            pl.BlockSpec((_BR, _BC), lambda c: (0, c)),
            pl.BlockSpec((_SUB, 2 * _SUB), lambda c: (0, 0)),
        ],
        out_specs=pl.BlockSpec((_BR, _BC), lambda c: (0, c)),
        out_shape=jax.ShapeDtypeStruct((R, C), x.dtype),
        scratch_shapes=[pltpu.VMEM((_BR, _SUB), jnp.float32)],
        compiler_params=pltpu.CompilerParams(
            dimension_semantics=("arbitrary",),
        ),
    )(x, m)


# speedup vs baseline: 1.0039x; 1.0039x over previous
import jax
import jax.numpy as jnp
import numpy as np
from jax.experimental import pallas as pl
from jax.experimental.pallas import tpu as pltpu

_BR = 1024
_BC = 2048
_SUB = 128
_K = _BC // _SUB

_M = np.concatenate(
    [np.triu(np.ones((_SUB, _SUB), np.float32)),
     np.ones((_SUB, _SUB), np.float32)], axis=1)


def _body(x_ref, m_ref, o_ref, carry_ref):
    c = pl.program_id(0)

    @pl.when(c == 0)
    def _():
        carry_ref[...] = jnp.zeros_like(carry_ref)

    t = x_ref[...]
    m = m_ref[...]
    off = carry_ref[...]
    for i in range(_K):
        sub = t[:, i * _SUB:(i + 1) * _SUB]
        r = jax.lax.dot(
            sub, m,
            precision=jax.lax.Precision.DEFAULT,
            preferred_element_type=jnp.float32,
        )
        o_ref[:, i * _SUB:(i + 1) * _SUB] = r[:, :_SUB] + off
        off = off + r[:, _SUB:]
    carry_ref[...] = off


@jax.jit
def kernel(x):
    R, C = x.shape
    m = jnp.asarray(_M)
    grid = (C // _BC,)
    return pl.pallas_call(
        _body,
        grid=grid,
        in_specs=[
            pl.BlockSpec((_BR, _BC), lambda c: (0, c)),
            pl.BlockSpec((_SUB, 2 * _SUB), lambda c: (0, 0)),
        ],
        out_specs=pl.BlockSpec((_BR, _BC), lambda c: (0, c)),
        out_shape=jax.ShapeDtypeStruct((R, C), x.dtype),
        scratch_shapes=[pltpu.VMEM((_BR, _SUB), jnp.float32)],
        compiler_params=pltpu.CompilerParams(
            dimension_semantics=("arbitrary",),
        ),
    )(x, m)
